# Initial kernel scaffold; baseline (speedup 1.0000x reference)
import jax
import jax.numpy as jnp
from jax.experimental import pallas as pl


def kernel(entity_emb, relation_emb, heads, relations, tails):
    raise NotImplementedError("write your pallas kernel here")


# speedup vs baseline: 2.5560x; 2.5560x over previous
import functools

import jax
import jax.numpy as jnp
from jax import lax
from jax.experimental import pallas as pl
from jax.experimental.pallas import tpu as pltpu
from jax.experimental.pallas import tpu_sc as plsc

_D = 128
_B = 16384
_NC = 2
_NS = 16
_NW = _NC * _NS
_BPW = _B // _NW
_C = 128
_NCHUNK = _BPW // _C
_L = 16


def _sqrt16(x):
    i = lax.bitcast_convert_type(x, jnp.int32)
    y = lax.bitcast_convert_type(0x5F3759DF - (i >> 1), jnp.float32)
    for _ in range(3):
        y = y * (1.5 - 0.5 * x * y * y)
    return x * y


def _tec_body(ent, rel, heads, rels, tails, out,
              idx_h, idx_r, idx_t,
              bh0, br0, bt0, bh1, br1, bt1,
              out_v, sem0, sem1):
    wid = lax.axis_index("s") * _NC + lax.axis_index("c")
    base = wid * _BPW

    pltpu.sync_copy(heads.at[pl.ds(base, _BPW)], idx_h)
    pltpu.sync_copy(rels.at[pl.ds(base, _BPW)], idx_r)
    pltpu.sync_copy(tails.at[pl.ds(base, _BPW)], idx_t)

    bufs = ((bh0, br0, bt0, sem0), (bh1, br1, bt1, sem1))

    def fire(c):
        bh, br, bt, sem = bufs[c % 2]
        off = c * _C
        return (
            pltpu.async_copy(ent.at[idx_h.at[pl.ds(off, _C)]], bh, sem),
            pltpu.async_copy(rel.at[idx_r.at[pl.ds(off, _C)]], br, sem),
            pltpu.async_copy(ent.at[idx_t.at[pl.ds(off, _C)]], bt, sem),
        )

    lane = lax.iota(jnp.int32, _L)
    pend = fire(0)
    for c in range(_NCHUNK):
        for hdl in pend:
            hdl.wait()
        if c + 1 < _NCHUNK:
            pend = fire(c + 1)
        bh, br, bt, _ = bufs[c % 2]

        def group_body(g, carry, bh=bh, br=br, bt=bt, off=c * _C):
            def row_ins(ii, vec):
                i = g * _L + ii
                acc = jnp.zeros((_L,), jnp.float32)
                for j in range(_D // _L):
                    sl = pl.ds(j * _L, _L)
                    s = (bh[i, sl] + br[i, sl]) - bt[i, sl]
                    acc = acc + s * s
                for sh in (8, 4, 2, 1):
                    acc = acc + acc.at[lane ^ sh].get(
                        mode="promise_in_bounds")
                return jnp.where(lane == ii, acc, vec)

            vec = lax.fori_loop(0, _L, row_ins, jnp.zeros((_L,), jnp.float32),
                                unroll=2)
            out_v[pl.ds(off + g * _L, _L)] = _sqrt16(vec)
            return carry

        lax.fori_loop(0, _C // _L, group_body, 0)

    pltpu.sync_copy(out_v, out.at[pl.ds(base, _BPW)])


_kg_call = functools.partial(
    pl.kernel,
    mesh=plsc.VectorSubcoreMesh(core_axis_name="c", subcore_axis_name="s"),
    out_type=jax.ShapeDtypeStruct((_B,), jnp.float32),
    scratch_types=[
        pltpu.VMEM((_BPW,), jnp.int32),
        pltpu.VMEM((_BPW,), jnp.int32),
        pltpu.VMEM((_BPW,), jnp.int32),
        pltpu.VMEM((_C, _D), jnp.float32),
        pltpu.VMEM((_C, _D), jnp.float32),
        pltpu.VMEM((_C, _D), jnp.float32),
        pltpu.VMEM((_C, _D), jnp.float32),
        pltpu.VMEM((_C, _D), jnp.float32),
        pltpu.VMEM((_C, _D), jnp.float32),
        pltpu.VMEM((_BPW,), jnp.float32),
        pltpu.SemaphoreType.DMA,
        pltpu.SemaphoreType.DMA,
    ],
)(_tec_body)


def kernel(entity_emb, relation_emb, heads, relations, tails):
    h = heads.astype(jnp.int32)
    r = relations.astype(jnp.int32)
    t = tails.astype(jnp.int32)
    return _kg_call(entity_emb, relation_emb, h, r, t)
